# baseline (speedup 1.0000x reference)
import jax
import jax.numpy as jnp
from jax import lax
from jax.experimental import pallas as pl
from jax.experimental.pallas import tpu as pltpu
from jax.experimental.pallas import tpu_sc as plsc

N_NODES = 10000
N_FEAT = 256
NFG = 16
WPN = N_FEAT // NFG // 2
NEG = 2
N_EDGES = 160000
EDGES_PER_EG = N_EDGES // NEG
CHUNK = 8000
NCHUNK = EDGES_PER_EG // CHUNK
GROUPS = CHUNK // 16


def _sc_body(zt_hbm, row_hbm, col_hbm, w_hbm, out_hbm,
             zt0, zt1, zt2, zt3, zt4, zt5, zt6, zt7,
             r0, c0, w0, r1, c1, w1, acc_v, sem0, sem1):
    ztiles = (zt0, zt1, zt2, zt3, zt4, zt5, zt6, zt7)
    wid = lax.axis_index("s") * 2 + lax.axis_index("c")
    fg = wid % NFG
    eg = wid // NFG
    for j in range(WPN):
        pltpu.sync_copy(zt_hbm.at[fg * WPN + j], ztiles[j])
    ebase = eg * EDGES_PER_EG

    bufs = ((r0, c0, w0, sem0), (r1, c1, w1, sem1))

    def start_chunk(ci):
        off = ebase + ci * CHUNK
        r, c, w, sem = bufs[ci % 2]
        return (pltpu.async_copy(row_hbm.at[pl.ds(off, CHUNK)], r, sem),
                pltpu.async_copy(col_hbm.at[pl.ds(off, CHUNK)], c, sem),
                pltpu.async_copy(w_hbm.at[pl.ds(off, CHUNK)], w, sem))

    def run_chunk(ci, acc0):
        r_v, c_v, w_v, _ = bufs[ci % 2]

        @plsc.parallel_loop(0, GROUPS, carry=acc0)
        def acc_out(g, acc):
            r16 = r_v[pl.ds(g * 16, 16)]
            c16 = c_v[pl.ds(g * 16, 16)]
            w16 = w_v[pl.ds(g * 16, 16)]
            w2 = w16 * w16
            slo = None
            shi = None
            for j in range(WPN):
                gr = plsc.load_gather(ztiles[j], [r16])
                gc = plsc.load_gather(ztiles[j], [c16])
                d = (plsc.bitcast(gr, jnp.bfloat16)
                     - plsc.bitcast(gc, jnp.bfloat16))
                d2 = plsc.bitcast(d * d, jnp.int32)
                d2lo = plsc.bitcast(d2 << 16, jnp.float32)
                d2hi = plsc.bitcast(d2 & jnp.int32(-65536), jnp.float32)
                slo = d2lo if slo is None else slo + d2lo
                shi = d2hi if shi is None else shi + d2hi
            return acc + w2 * (slo + shi)

        return acc_out

    acc = jnp.zeros((16,), jnp.float32)
    descs = start_chunk(0)
    for ci in range(NCHUNK):
        nxt = start_chunk(ci + 1) if ci + 1 < NCHUNK else ()
        for dsc in descs:
            dsc.wait()
        acc = run_chunk(ci, acc)
        descs = nxt

    acc_v[...] = acc
    pltpu.sync_copy(acc_v, out_hbm.at[wid])


_sc_call = pl.kernel(
    _sc_body,
    out_type=jax.ShapeDtypeStruct((NEG * NFG, 16), jnp.float32),
    mesh=plsc.VectorSubcoreMesh(core_axis_name="c", subcore_axis_name="s"),
    scratch_types=[
        pltpu.VMEM((N_NODES,), jnp.int32),
        pltpu.VMEM((N_NODES,), jnp.int32),
        pltpu.VMEM((N_NODES,), jnp.int32),
        pltpu.VMEM((N_NODES,), jnp.int32),
        pltpu.VMEM((N_NODES,), jnp.int32),
        pltpu.VMEM((N_NODES,), jnp.int32),
        pltpu.VMEM((N_NODES,), jnp.int32),
        pltpu.VMEM((N_NODES,), jnp.int32),
        pltpu.VMEM((CHUNK,), jnp.int32),
        pltpu.VMEM((CHUNK,), jnp.int32),
        pltpu.VMEM((CHUNK,), jnp.float32),
        pltpu.VMEM((CHUNK,), jnp.int32),
        pltpu.VMEM((CHUNK,), jnp.int32),
        pltpu.VMEM((CHUNK,), jnp.float32),
        pltpu.VMEM((16,), jnp.float32),
        pltpu.SemaphoreType.DMA,
        pltpu.SemaphoreType.DMA,
    ],
    compiler_params=pltpu.CompilerParams(needs_layout_passes=False),
)


def kernel(z, edge_index, edge_weight):
    row = edge_index[0].astype(jnp.int32)
    col = edge_index[1].astype(jnp.int32)
    zbf = z.astype(jnp.bfloat16).reshape(N_NODES, N_FEAT // 2, 2)
    zp = jax.lax.bitcast_convert_type(zbf, jnp.int32)
    zp = jax.lax.optimization_barrier(zp)
    zt = zp.T
    partials = _sc_call(zt, row, col, edge_weight)
    return jnp.sum(partials) / edge_index.shape[1]


# speedup vs baseline: 1.0020x; 1.0020x over previous
import jax
import jax.numpy as jnp
from jax import lax
from jax.experimental import pallas as pl
from jax.experimental.pallas import tpu as pltpu
from jax.experimental.pallas import tpu_sc as plsc

N_NODES = 10000
N_FEAT = 256
NWORDS = N_FEAT // 2
NTILES = 32
EDGES_PER_TILE = 5000
CHUNK = 128
NFULL = EDGES_PER_TILE // CHUNK
TAIL = EDGES_PER_TILE - NFULL * CHUNK

_MASKHI = -65536


def _edge_loop(n_edges, rr_v, rc_v, w_v, acc0):
    @plsc.parallel_loop(0, n_edges, carry=acc0)
    def acc_out(e, acc):
        esplat = jnp.zeros((16,), jnp.int32) + e
        wv = plsc.load_gather(w_v, [esplat])
        w2 = wv * wv
        slo = None
        shi = None
        for p in range(8):
            a = rr_v[e, pl.ds(p * 16, 16)]
            b = rc_v[e, pl.ds(p * 16, 16)]
            d = (plsc.bitcast(a, jnp.bfloat16)
                 - plsc.bitcast(b, jnp.bfloat16))
            d2 = plsc.bitcast(d * d, jnp.int32)
            d2lo = plsc.bitcast(d2 << 16, jnp.float32)
            d2hi = plsc.bitcast(d2 & _MASKHI, jnp.float32)
            slo = d2lo if slo is None else slo + d2lo
            shi = d2hi if shi is None else shi + d2hi
        return acc + w2 * (slo + shi)

    return acc_out


def _sc_body(zp_hbm, row_hbm, col_hbm, w_hbm, out_hbm,
             ir0, ic0, w0, rr0, rc0, ir1, ic1, w1, rr1, rc1,
             irt, ict, wt, rrt, rct, acc_v,
             semi0, semi1, semg0, semg1, semt):
    wid = lax.axis_index("s") * 2 + lax.axis_index("c")
    tbase = wid * EDGES_PER_TILE

    bufs = ((ir0, ic0, w0, rr0, rc0, semi0, semg0),
            (ir1, ic1, w1, rr1, rc1, semi1, semg1))

    def start_idx(ci):
        off = tbase + ci * CHUNK
        ir, ic, w, _, _, semi, _ = bufs[ci % 2]
        return (pltpu.async_copy(row_hbm.at[pl.ds(off, CHUNK)], ir, semi),
                pltpu.async_copy(col_hbm.at[pl.ds(off, CHUNK)], ic, semi),
                pltpu.async_copy(w_hbm.at[pl.ds(off, CHUNK)], w, semi))

    def start_gather(ci):
        ir, ic, _, rr, rc, _, semg = bufs[ci % 2]
        return (pltpu.async_copy(zp_hbm.at[ir], rr, semg),
                pltpu.async_copy(zp_hbm.at[ic], rc, semg))

    acc = jnp.zeros((16,), jnp.float32)

    idx_d = [None] * (NFULL + 1)
    gat_d = [None] * NFULL
    idx_d[0] = start_idx(0)
    for dsc in idx_d[0]:
        dsc.wait()
    gat_d[0] = start_gather(0)
    if NFULL > 1:
        idx_d[1] = start_idx(1)
    for ci in range(NFULL):
        for dsc in gat_d[ci]:
            dsc.wait()
        if ci + 1 < NFULL:
            for dsc in idx_d[ci + 1]:
                dsc.wait()
            gat_d[ci + 1] = start_gather(ci + 1)
        _, _, w_v, rr_v, rc_v, _, _ = bufs[ci % 2]
        acc = _edge_loop(CHUNK, rr_v, rc_v, w_v, acc)
        if ci + 2 < NFULL:
            idx_d[ci + 2] = start_idx(ci + 2)

    toff = tbase + NFULL * CHUNK
    pltpu.async_copy(row_hbm.at[pl.ds(toff, TAIL)], irt, semt).wait()
    pltpu.async_copy(col_hbm.at[pl.ds(toff, TAIL)], ict, semt).wait()
    pltpu.async_copy(w_hbm.at[pl.ds(toff, TAIL)], wt, semt).wait()
    pltpu.async_copy(zp_hbm.at[irt], rrt, semt).wait()
    pltpu.async_copy(zp_hbm.at[ict], rct, semt).wait()
    acc = _edge_loop(TAIL, rrt, rct, wt, acc)

    acc_v[...] = acc
    pltpu.sync_copy(acc_v, out_hbm.at[wid])


_sc_call = pl.kernel(
    _sc_body,
    out_type=jax.ShapeDtypeStruct((NTILES, 16), jnp.float32),
    mesh=plsc.VectorSubcoreMesh(core_axis_name="c", subcore_axis_name="s"),
    scratch_types=[
        pltpu.VMEM((CHUNK,), jnp.int32),
        pltpu.VMEM((CHUNK,), jnp.int32),
        pltpu.VMEM((CHUNK,), jnp.float32),
        pltpu.VMEM((CHUNK, NWORDS), jnp.int32),
        pltpu.VMEM((CHUNK, NWORDS), jnp.int32),
        pltpu.VMEM((CHUNK,), jnp.int32),
        pltpu.VMEM((CHUNK,), jnp.int32),
        pltpu.VMEM((CHUNK,), jnp.float32),
        pltpu.VMEM((CHUNK, NWORDS), jnp.int32),
        pltpu.VMEM((CHUNK, NWORDS), jnp.int32),
        pltpu.VMEM((TAIL,), jnp.int32),
        pltpu.VMEM((TAIL,), jnp.int32),
        pltpu.VMEM((TAIL,), jnp.float32),
        pltpu.VMEM((TAIL, NWORDS), jnp.int32),
        pltpu.VMEM((TAIL, NWORDS), jnp.int32),
        pltpu.VMEM((16,), jnp.float32),
        pltpu.SemaphoreType.DMA,
        pltpu.SemaphoreType.DMA,
        pltpu.SemaphoreType.DMA,
        pltpu.SemaphoreType.DMA,
        pltpu.SemaphoreType.DMA,
    ],
    compiler_params=pltpu.CompilerParams(needs_layout_passes=False),
)


def kernel(z, edge_index, edge_weight):
    row = edge_index[0].astype(jnp.int32)
    col = edge_index[1].astype(jnp.int32)
    zbf = z.astype(jnp.bfloat16).reshape(N_NODES, NWORDS, 2)
    zp = jax.lax.bitcast_convert_type(zbf, jnp.int32)
    partials = _sc_call(zp, row, col, edge_weight)
    return jnp.sum(partials) / edge_index.shape[1]


# speedup vs baseline: 1.7560x; 1.7526x over previous
import jax
import jax.numpy as jnp
from jax import lax
from jax.experimental import pallas as pl
from jax.experimental.pallas import tpu as pltpu
from jax.experimental.pallas import tpu_sc as plsc

N_NODES = 10000
N_FEAT = 256
NWORDS = N_FEAT // 2
NTILES = 32
EDGES_PER_TILE = 5000
CHUNK = 128
NFULL = EDGES_PER_TILE // CHUNK
TAIL = EDGES_PER_TILE - NFULL * CHUNK

_MASKHI = -65536


def _edge_loop(n_edges, rr_v, rc_v, w_v, acc0):
    @plsc.parallel_loop(0, n_edges, carry=acc0)
    def acc_out(e, acc):
        esplat = jnp.zeros((16,), jnp.int32) + e
        wv = plsc.load_gather(w_v, [esplat])
        w2 = wv * wv
        slo = None
        shi = None
        for p in range(8):
            a = rr_v[e, pl.ds(p * 16, 16)]
            b = rc_v[e, pl.ds(p * 16, 16)]
            d = (plsc.bitcast(a, jnp.bfloat16)
                 - plsc.bitcast(b, jnp.bfloat16))
            d2 = plsc.bitcast(d * d, jnp.int32)
            d2lo = plsc.bitcast(d2 << 16, jnp.float32)
            d2hi = plsc.bitcast(d2 & _MASKHI, jnp.float32)
            slo = d2lo if slo is None else slo + d2lo
            shi = d2hi if shi is None else shi + d2hi
        return acc + w2 * (slo + shi)

    return acc_out


def _sc_body(zp_hbm, row_hbm, col_hbm, w_hbm, out_hbm,
             ir0, ic0, w0, rr0, rc0, ir1, ic1, w1, rr1, rc1,
             irt, ict, wt, rrt, rct, acc_v,
             semi0, semi1, semg0, semg1, semt):
    wid = lax.axis_index("s") * 2 + lax.axis_index("c")
    tbase = wid * EDGES_PER_TILE

    bufs = ((ir0, ic0, w0, rr0, rc0, semi0, semg0),
            (ir1, ic1, w1, rr1, rc1, semi1, semg1))

    def start_idx(ci):
        off = tbase + ci * CHUNK
        ir, ic, w, _, _, semi, _ = bufs[ci % 2]
        return (pltpu.async_copy(row_hbm.at[pl.ds(off, CHUNK)], ir, semi),
                pltpu.async_copy(col_hbm.at[pl.ds(off, CHUNK)], ic, semi),
                pltpu.async_copy(w_hbm.at[pl.ds(off, CHUNK)], w, semi))

    def start_gather(ci):
        ir, ic, _, rr, rc, _, semg = bufs[ci % 2]
        return (pltpu.async_copy(zp_hbm.at[ir], rr, semg),
                pltpu.async_copy(zp_hbm.at[ic], rc, semg))

    acc = jnp.zeros((16,), jnp.float32)

    idx_d = [None] * (NFULL + 1)
    gat_d = [None] * NFULL
    idx_d[0] = start_idx(0)
    for dsc in idx_d[0]:
        dsc.wait()
    gat_d[0] = start_gather(0)
    if NFULL > 1:
        idx_d[1] = start_idx(1)
    for ci in range(NFULL):
        for dsc in gat_d[ci]:
            dsc.wait()
        if ci + 1 < NFULL:
            for dsc in idx_d[ci + 1]:
                dsc.wait()
            gat_d[ci + 1] = start_gather(ci + 1)
        _, _, w_v, rr_v, rc_v, _, _ = bufs[ci % 2]
        acc = _edge_loop(CHUNK, rr_v, rc_v, w_v, acc)
        if ci + 2 < NFULL:
            idx_d[ci + 2] = start_idx(ci + 2)

    toff = tbase + NFULL * CHUNK
    pltpu.async_copy(row_hbm.at[pl.ds(toff, TAIL)], irt, semt).wait()
    pltpu.async_copy(col_hbm.at[pl.ds(toff, TAIL)], ict, semt).wait()
    pltpu.async_copy(w_hbm.at[pl.ds(toff, TAIL)], wt, semt).wait()
    pltpu.async_copy(zp_hbm.at[irt], rrt, semt).wait()
    pltpu.async_copy(zp_hbm.at[ict], rct, semt).wait()
    acc = _edge_loop(TAIL, rrt, rct, wt, acc)

    acc_v[...] = acc
    pltpu.sync_copy(acc_v, out_hbm.at[wid])


_sc_call = pl.kernel(
    _sc_body,
    out_type=jax.ShapeDtypeStruct((NTILES, 16), jnp.float32),
    mesh=plsc.VectorSubcoreMesh(core_axis_name="c", subcore_axis_name="s"),
    scratch_types=[
        pltpu.VMEM((CHUNK,), jnp.int32),
        pltpu.VMEM((CHUNK,), jnp.int32),
        pltpu.VMEM((CHUNK,), jnp.float32),
        pltpu.VMEM((CHUNK, NWORDS), jnp.int32),
        pltpu.VMEM((CHUNK, NWORDS), jnp.int32),
        pltpu.VMEM((CHUNK,), jnp.int32),
        pltpu.VMEM((CHUNK,), jnp.int32),
        pltpu.VMEM((CHUNK,), jnp.float32),
        pltpu.VMEM((CHUNK, NWORDS), jnp.int32),
        pltpu.VMEM((CHUNK, NWORDS), jnp.int32),
        pltpu.VMEM((TAIL,), jnp.int32),
        pltpu.VMEM((TAIL,), jnp.int32),
        pltpu.VMEM((TAIL,), jnp.float32),
        pltpu.VMEM((TAIL, NWORDS), jnp.int32),
        pltpu.VMEM((TAIL, NWORDS), jnp.int32),
        pltpu.VMEM((16,), jnp.float32),
        pltpu.SemaphoreType.DMA,
        pltpu.SemaphoreType.DMA,
        pltpu.SemaphoreType.DMA,
        pltpu.SemaphoreType.DMA,
        pltpu.SemaphoreType.DMA,
    ],
    compiler_params=pltpu.CompilerParams(needs_layout_passes=False),
)


def kernel(z, edge_index, edge_weight):
    row = edge_index[0].astype(jnp.int32)
    col = edge_index[1].astype(jnp.int32)
    zi = jax.lax.bitcast_convert_type(z, jnp.int32)
    a = zi[:, :NWORDS] + 0x8000
    b = zi[:, NWORDS:] + 0x8000
    zp = ((a >> 16) & 0xFFFF) | (b & -65536)
    partials = _sc_call(zp, row, col, edge_weight)
    return jnp.sum(partials) / edge_index.shape[1]


# speedup vs baseline: 1.8216x; 1.0373x over previous
import jax
import jax.numpy as jnp
from jax import lax
from jax.experimental import pallas as pl
from jax.experimental.pallas import tpu as pltpu
from jax.experimental.pallas import tpu_sc as plsc

N_NODES = 10000
N_FEAT = 256
N_EDGES = 160000
NWORDS = N_FEAT // 2
NTILES = 32
EDGES_PER_TILE = 5000
CHUNK = 128
NFULL = EDGES_PER_TILE // CHUNK
TAIL = EDGES_PER_TILE - NFULL * CHUNK

_MASKHI = -65536


def _edge_loop(n_edges, rr_v, rc_v, w_v, acc0):
    @plsc.parallel_loop(0, n_edges, carry=acc0)
    def acc_out(e, acc):
        esplat = jnp.zeros((16,), jnp.int32) + e
        wv = plsc.load_gather(w_v, [esplat])
        w2 = wv * wv
        slo = None
        shi = None
        for p in range(8):
            a = rr_v[e, pl.ds(p * 16, 16)]
            b = rc_v[e, pl.ds(p * 16, 16)]
            d = (plsc.bitcast(a, jnp.bfloat16)
                 - plsc.bitcast(b, jnp.bfloat16))
            d2 = plsc.bitcast(d * d, jnp.int32)
            d2lo = plsc.bitcast(d2 << 16, jnp.float32)
            d2hi = plsc.bitcast(d2 & _MASKHI, jnp.float32)
            slo = d2lo if slo is None else slo + d2lo
            shi = d2hi if shi is None else shi + d2hi
        return acc + w2 * (slo + shi)

    return acc_out


def _sc_body(zp_hbm, ei_hbm, w_hbm, out_hbm,
             ir0, ic0, w0, rr0, rc0, ir1, ic1, w1, rr1, rc1,
             irt, ict, wt, rrt, rct, acc_v,
             semi0, semi1, semg0, semg1, semt):
    wid = lax.axis_index("s") * 2 + lax.axis_index("c")
    tbase = wid * EDGES_PER_TILE

    bufs = ((ir0, ic0, w0, rr0, rc0, semi0, semg0),
            (ir1, ic1, w1, rr1, rc1, semi1, semg1))

    def start_idx(ci):
        off = tbase + ci * CHUNK
        ir, ic, w, _, _, semi, _ = bufs[ci % 2]
        return (pltpu.async_copy(ei_hbm.at[pl.ds(off, CHUNK)], ir, semi),
                pltpu.async_copy(ei_hbm.at[pl.ds(N_EDGES + off, CHUNK)], ic, semi),
                pltpu.async_copy(w_hbm.at[pl.ds(off, CHUNK)], w, semi))

    def start_gather(ci):
        ir, ic, _, rr, rc, _, semg = bufs[ci % 2]
        return (pltpu.async_copy(zp_hbm.at[ir], rr, semg),
                pltpu.async_copy(zp_hbm.at[ic], rc, semg))

    acc = jnp.zeros((16,), jnp.float32)

    idx_d = [None] * (NFULL + 1)
    gat_d = [None] * NFULL
    idx_d[0] = start_idx(0)
    for dsc in idx_d[0]:
        dsc.wait()
    gat_d[0] = start_gather(0)
    if NFULL > 1:
        idx_d[1] = start_idx(1)
    for ci in range(NFULL):
        for dsc in gat_d[ci]:
            dsc.wait()
        if ci + 1 < NFULL:
            for dsc in idx_d[ci + 1]:
                dsc.wait()
            gat_d[ci + 1] = start_gather(ci + 1)
        _, _, w_v, rr_v, rc_v, _, _ = bufs[ci % 2]
        acc = _edge_loop(CHUNK, rr_v, rc_v, w_v, acc)
        if ci + 2 < NFULL:
            idx_d[ci + 2] = start_idx(ci + 2)

    toff = tbase + NFULL * CHUNK
    pltpu.async_copy(ei_hbm.at[pl.ds(toff, TAIL)], irt, semt).wait()
    pltpu.async_copy(ei_hbm.at[pl.ds(N_EDGES + toff, TAIL)], ict, semt).wait()
    pltpu.async_copy(w_hbm.at[pl.ds(toff, TAIL)], wt, semt).wait()
    pltpu.async_copy(zp_hbm.at[irt], rrt, semt).wait()
    pltpu.async_copy(zp_hbm.at[ict], rct, semt).wait()
    acc = _edge_loop(TAIL, rrt, rct, wt, acc)

    acc_v[...] = acc
    pltpu.sync_copy(acc_v, out_hbm.at[wid])


_sc_call = pl.kernel(
    _sc_body,
    out_type=jax.ShapeDtypeStruct((NTILES, 16), jnp.float32),
    mesh=plsc.VectorSubcoreMesh(core_axis_name="c", subcore_axis_name="s"),
    scratch_types=[
        pltpu.VMEM((CHUNK,), jnp.int32),
        pltpu.VMEM((CHUNK,), jnp.int32),
        pltpu.VMEM((CHUNK,), jnp.float32),
        pltpu.VMEM((CHUNK, NWORDS), jnp.int32),
        pltpu.VMEM((CHUNK, NWORDS), jnp.int32),
        pltpu.VMEM((CHUNK,), jnp.int32),
        pltpu.VMEM((CHUNK,), jnp.int32),
        pltpu.VMEM((CHUNK,), jnp.float32),
        pltpu.VMEM((CHUNK, NWORDS), jnp.int32),
        pltpu.VMEM((CHUNK, NWORDS), jnp.int32),
        pltpu.VMEM((TAIL,), jnp.int32),
        pltpu.VMEM((TAIL,), jnp.int32),
        pltpu.VMEM((TAIL,), jnp.float32),
        pltpu.VMEM((TAIL, NWORDS), jnp.int32),
        pltpu.VMEM((TAIL, NWORDS), jnp.int32),
        pltpu.VMEM((16,), jnp.float32),
        pltpu.SemaphoreType.DMA,
        pltpu.SemaphoreType.DMA,
        pltpu.SemaphoreType.DMA,
        pltpu.SemaphoreType.DMA,
        pltpu.SemaphoreType.DMA,
    ],
    compiler_params=pltpu.CompilerParams(needs_layout_passes=False),
)


def kernel(z, edge_index, edge_weight):
    ei = edge_index.astype(jnp.int32).reshape(-1)
    a = jax.lax.bitcast_convert_type(z[:, :NWORDS], jnp.int32) + 0x8000
    b = jax.lax.bitcast_convert_type(z[:, NWORDS:], jnp.int32) + 0x8000
    zp = ((a >> 16) & 0xFFFF) | (b & -65536)
    partials = _sc_call(zp, ei, edge_weight)
    return jnp.sum(partials) / edge_index.shape[1]
